# speedup vs baseline: 1.3640x; 1.3640x over previous
import functools

import numpy as np

import jax
import jax.numpy as jnp
from jax.experimental import pallas as pl
from jax.experimental.pallas import tpu as pltpu


_VMEM_LIMIT = 48 * 1024 * 1024


def _hann_padded(win_length, n_fft):
    n = np.arange(win_length)
    w = 0.5 - 0.5 * np.cos(2.0 * np.pi * n / win_length)
    out = np.zeros(n_fft, dtype=np.float64)
    lpad = (n_fft - win_length) // 2
    out[lpad:lpad + win_length] = w
    return out


@functools.lru_cache(maxsize=None)
def _host_tables(n_fft, win_length, hop, T, p0, nb_out):
    F = n_fft // 2 + 1
    win = _hann_padded(win_length, n_fft)
    f = np.arange(F, dtype=np.float64)[:, None]
    o = np.arange(n_fft, dtype=np.float64)[None, :]
    ang = (2.0 * np.pi / n_fft) * f * o
    dup = np.ones((F, 1))
    dup[1:F - 1] = 2.0
    scale = win[None, :] / n_fft
    A = (dup * np.cos(ang)) * scale
    Bm = (-dup * np.sin(ang)) * scale

    win_sq = win ** 2
    n_samples = (T - 1) * hop + n_fft
    wsum = np.zeros(n_samples, dtype=np.float64)
    for t in range(T):
        wsum[t * hop:t * hop + n_fft] += win_sq
    inv = 1.0 / np.clip(wsum, 1e-11, None)
    inv_blocks = inv.reshape(-1, hop)[p0:p0 + nb_out].astype(np.float32)

    to_bf16 = lambda m: jnp.asarray(m.astype(np.float32), dtype=jnp.bfloat16)
    return to_bf16(A), to_bf16(Bm), jnp.asarray(inv_blocks)


def _fused_kernel(re_ref, im_ref, a_ref, b_ref, inv_ref, o_ref, acc_ref,
                  *, ratio, T, p0, nb_out, hop):
    fr = jnp.dot(re_ref[0].astype(jnp.bfloat16), a_ref[...],
                 preferred_element_type=jnp.float32)
    fr = fr + jnp.dot(im_ref[0].astype(jnp.bfloat16), b_ref[...],
                      preferred_element_type=jnp.float32)
    acc_ref[...] = jnp.zeros_like(acc_ref)
    for k in range(ratio):
        acc_ref[k:k + T, :] += fr[:, k * hop:(k + 1) * hop]
    o_ref[0] = acc_ref[p0:p0 + nb_out, :] * inv_ref[...]


def _fused_istft(re, im, *, n_fft, hop, length):
    B, T, F = re.shape
    assert F == n_fft // 2 + 1
    ratio = n_fft // hop
    start = n_fft // 2
    assert start % hop == 0 and length % hop == 0
    p0 = start // hop
    nb_out = length // hop
    A, Bm, inv_blocks = _host_tables(n_fft, n_fft, hop, T, p0, nb_out)

    body = functools.partial(_fused_kernel, ratio=ratio, T=T, p0=p0,
                             nb_out=nb_out, hop=hop)
    y = pl.pallas_call(
        body,
        out_shape=jax.ShapeDtypeStruct((B, nb_out, hop), jnp.float32),
        grid=(B,),
        in_specs=[
            pl.BlockSpec((1, T, F), lambda b: (b, 0, 0)),
            pl.BlockSpec((1, T, F), lambda b: (b, 0, 0)),
            pl.BlockSpec((F, n_fft), lambda b: (0, 0)),
            pl.BlockSpec((F, n_fft), lambda b: (0, 0)),
            pl.BlockSpec((nb_out, hop), lambda b: (0, 0)),
        ],
        out_specs=pl.BlockSpec((1, nb_out, hop), lambda b: (b, 0, 0)),
        scratch_shapes=[pltpu.VMEM((T + ratio - 1, hop), jnp.float32)],
        compiler_params=pltpu.CompilerParams(
            dimension_semantics=("parallel",),
            vmem_limit_bytes=_VMEM_LIMIT,
        ),
    )(re, im, A, Bm, inv_blocks)
    return y.reshape(B, length)


def kernel(real_stft, imag_stft):
    return _fused_istft(real_stft[:, 0], imag_stft[:, 0],
                        n_fft=2048, hop=512, length=261632)
